# baseline (speedup 1.0000x reference)
import functools

import jax
import jax.numpy as jnp
from jax import lax
from jax.experimental import pallas as pl
from jax.experimental.pallas import tpu as pltpu
from jax.experimental.pallas import tpu_sc as plsc

NC, NS, L = 2, 16, 16
NW = NC * NS
B, C, N = 128, 512, 128
HALF = N // 2
ROWS = B * C
RPW = ROWS // NW
CHUNK = 256
NCHUNK = RPW // CHUNK

_mesh = plsc.VectorSubcoreMesh(core_axis_name="c", subcore_axis_name="s")


@functools.partial(
    pl.kernel,
    mesh=_mesh,
    out_type=jax.ShapeDtypeStruct((ROWS * HALF,), jnp.float32),
    scratch_types=[
        pltpu.VMEM((CHUNK * N,), jnp.float32),
        pltpu.VMEM((CHUNK * N,), jnp.float32),
        pltpu.VMEM((CHUNK * HALF,), jnp.float32),
        pltpu.VMEM((CHUNK * HALF,), jnp.float32),
        pltpu.SemaphoreType.DMA,
        pltpu.SemaphoreType.DMA,
        pltpu.SemaphoreType.DMA,
        pltpu.SemaphoreType.DMA,
    ],
)
def _sc_maxpool(x_hbm, o_hbm, xv0, xv1, ov0, ov1, is0, is1, os0, os1):
    wid = lax.axis_index("s") * NC + lax.axis_index("c")
    base_row = wid * RPW
    xb, ob, isem, osem = (xv0, xv1), (ov0, ov1), (is0, is1), (os0, os1)

    def start_in(ci):
        b = ci % 2
        return pltpu.async_copy(
            x_hbm.at[pl.ds((base_row + ci * CHUNK) * N, CHUNK * N)],
            xb[b], isem[b])

    def start_out(ci):
        b = ci % 2
        return pltpu.async_copy(
            ob[b],
            o_hbm.at[pl.ds((base_row + ci * CHUNK) * HALF, CHUNK * HALF)],
            osem[b])

    def compute(ci):
        xv, ov = xb[ci % 2], ob[ci % 2]

        RPB = 16

        def row_body(rb, c2):
            base = rb * (RPB * N)
            obase = rb * (RPB * HALF)
            for rr in range(RPB):
                for kk in range(HALF // L):
                    a = xv[pl.ds(base + rr * N + kk * L, L)]
                    b2 = xv[pl.ds(base + rr * N + HALF + kk * L, L)]
                    ov[pl.ds(obase + rr * HALF + kk * L, L)] = jnp.maximum(a, b2)
            return c2

        lax.fori_loop(0, CHUNK // RPB, row_body, 0)

    in_d = {0: start_in(0), 1: start_in(1)}
    out_d = {}
    for ci in range(NCHUNK):
        in_d[ci].wait()
        if ci >= 2:
            out_d[ci - 2].wait()
        compute(ci)
        out_d[ci] = start_out(ci)
        if ci + 2 < NCHUNK:
            in_d[ci + 2] = start_in(ci + 2)
    out_d[NCHUNK - 2].wait()
    out_d[NCHUNK - 1].wait()


def kernel(data):
    x = data.reshape(ROWS * N)
    out = _sc_maxpool(x)
    return out.reshape(B, C, HALF)


# speedup vs baseline: 1.2882x; 1.2882x over previous
import functools

import jax
import jax.numpy as jnp
from jax import lax
from jax.experimental import pallas as pl
from jax.experimental.pallas import tpu as pltpu
from jax.experimental.pallas import tpu_sc as plsc

NC, NS, L = 2, 16, 16
NW = NC * NS
B, C, N = 128, 512, 128
HALF = N // 2
ROWS = B * C
RPW = ROWS // NW
CHUNK = 256
NCHUNK = RPW // CHUNK

_mesh = plsc.VectorSubcoreMesh(core_axis_name="c", subcore_axis_name="s")


@functools.partial(
    pl.kernel,
    mesh=_mesh,
    out_type=jax.ShapeDtypeStruct((ROWS * HALF,), jnp.float32),
    scratch_types=[
        pltpu.VMEM((CHUNK * N,), jnp.float32),
        pltpu.VMEM((CHUNK * N,), jnp.float32),
        pltpu.VMEM((CHUNK * HALF,), jnp.float32),
        pltpu.VMEM((CHUNK * HALF,), jnp.float32),
        pltpu.SemaphoreType.DMA,
        pltpu.SemaphoreType.DMA,
        pltpu.SemaphoreType.DMA,
        pltpu.SemaphoreType.DMA,
    ],
)
def _sc_maxpool(x_hbm, o_hbm, xv0, xv1, ov0, ov1, is0, is1, os0, os1):
    wid = lax.axis_index("s") * NC + lax.axis_index("c")
    base_row = wid * RPW
    xb, ob, isem, osem = (xv0, xv1), (ov0, ov1), (is0, is1), (os0, os1)

    def start_in(ci):
        b = ci % 2
        return pltpu.async_copy(
            x_hbm.at[pl.ds((base_row + ci * CHUNK) * N, CHUNK * N)],
            xb[b], isem[b])

    def start_out(ci):
        b = ci % 2
        return pltpu.async_copy(
            ob[b],
            o_hbm.at[pl.ds((base_row + ci * CHUNK) * HALF, CHUNK * HALF)],
            osem[b])

    def compute(ci):
        xv, ov = xb[ci % 2], ob[ci % 2]

        @plsc.parallel_loop(0, CHUNK, 1, unroll=4)
        def row_body(r):
            for kk in range(HALF // L):
                a = xv[pl.ds(r * N + kk * L, L)]
                b2 = xv[pl.ds(r * N + HALF + kk * L, L)]
                ov[pl.ds(r * HALF + kk * L, L)] = jnp.maximum(a, b2)

    in_d = {0: start_in(0), 1: start_in(1)}
    out_d = {}
    for ci in range(NCHUNK):
        in_d[ci].wait()
        if ci >= 2:
            out_d[ci - 2].wait()
        plsc.subcore_barrier()
        compute(ci)
        plsc.subcore_barrier()
        out_d[ci] = start_out(ci)
        if ci + 2 < NCHUNK:
            in_d[ci + 2] = start_in(ci + 2)
    out_d[NCHUNK - 2].wait()
    out_d[NCHUNK - 1].wait()


def kernel(data):
    x = data.reshape(ROWS * N)
    out = _sc_maxpool(x)
    return out.reshape(B, C, HALF)


# speedup vs baseline: 1.5594x; 1.2106x over previous
import functools

import jax
import jax.numpy as jnp
from jax import lax
from jax.experimental import pallas as pl
from jax.experimental.pallas import tpu as pltpu
from jax.experimental.pallas import tpu_sc as plsc

NC, NS, L = 2, 16, 16
NW = NC * NS
B, C, N = 128, 512, 128
HALF = N // 2
BPW = B // NW
CHUNK = 256
CPB = C // CHUNK
NCHUNK = BPW * CPB

_mesh = plsc.VectorSubcoreMesh(core_axis_name="c", subcore_axis_name="s")


@functools.partial(
    pl.kernel,
    mesh=_mesh,
    out_type=jax.ShapeDtypeStruct((B, C, HALF), jnp.float32),
    scratch_types=[
        pltpu.VMEM((CHUNK, N), jnp.float32),
        pltpu.VMEM((CHUNK, N), jnp.float32),
        pltpu.VMEM((CHUNK, HALF), jnp.float32),
        pltpu.VMEM((CHUNK, HALF), jnp.float32),
        pltpu.SemaphoreType.DMA,
        pltpu.SemaphoreType.DMA,
        pltpu.SemaphoreType.DMA,
        pltpu.SemaphoreType.DMA,
    ],
)
def _sc_maxpool(x_hbm, o_hbm, xv0, xv1, ov0, ov1, is0, is1, os0, os1):
    wid = lax.axis_index("s") * NC + lax.axis_index("c")
    b_base = wid * BPW
    xb, ob, isem, osem = (xv0, xv1), (ov0, ov1), (is0, is1), (os0, os1)

    def start_in(ci):
        b = b_base + ci // CPB
        c0 = (ci % CPB) * CHUNK
        return pltpu.async_copy(
            x_hbm.at[b, pl.ds(c0, CHUNK), :], xb[ci % 2], isem[ci % 2])

    def start_out(ci):
        b = b_base + ci // CPB
        c0 = (ci % CPB) * CHUNK
        return pltpu.async_copy(
            ob[ci % 2], o_hbm.at[b, pl.ds(c0, CHUNK), :], osem[ci % 2])

    def compute(ci):
        xv, ov = xb[ci % 2], ob[ci % 2]

        @plsc.parallel_loop(0, CHUNK, 1, unroll=4)
        def row_body(r):
            for kk in range(HALF // L):
                a = xv[r, pl.ds(kk * L, L)]
                b2 = xv[r, pl.ds(HALF + kk * L, L)]
                ov[r, pl.ds(kk * L, L)] = jnp.maximum(a, b2)

    in_d = {0: start_in(0), 1: start_in(1)}
    out_d = {}
    for ci in range(NCHUNK):
        in_d[ci].wait()
        if ci >= 2:
            out_d[ci - 2].wait()
        plsc.subcore_barrier()
        compute(ci)
        plsc.subcore_barrier()
        out_d[ci] = start_out(ci)
        if ci + 2 < NCHUNK:
            in_d[ci + 2] = start_in(ci + 2)
    out_d[NCHUNK - 2].wait()
    out_d[NCHUNK - 1].wait()


def kernel(data):
    return _sc_maxpool(data)
